# Initial kernel scaffold; baseline (speedup 1.0000x reference)
import jax
import jax.numpy as jnp
from jax.experimental import pallas as pl


def kernel(F, edge_index, w0_values, w1_values, d, mask):
    raise NotImplementedError("write your pallas kernel here")


# speedup vs baseline: 3.0307x; 3.0307x over previous
import functools

import jax
import jax.numpy as jnp
from jax import lax
from jax.experimental import pallas as pl
from jax.experimental.pallas import tpu as pltpu
from jax.experimental.pallas import tpu_sc as plsc

N = 10000
E = 320000
DF = 128
GAMMA = 1.0
STEPS = 14

NCORES = 2
NTILES = 16
CHUNK = 128
EPT = ((E + NTILES * CHUNK - 1) // (NTILES * CHUNK)) * CHUNK
E_PAD = EPT * NTILES
NCHUNKS = EPT // CHUNK
ROWS_PT = 640
ACC_N = NTILES * ROWS_PT
LAST_ROWS = N - 15 * ROWS_PT

_mesh = plsc.VectorSubcoreMesh(core_axis_name="c", subcore_axis_name="s")


@functools.partial(
    pl.kernel,
    out_type=(
        jax.ShapeDtypeStruct((N, DF), jnp.float32),
        jax.ShapeDtypeStruct((N, DF), jnp.float32),
    ),
    mesh=_mesh,
    scratch_types=[
        pltpu.VMEM_SHARED((ACC_N, DF), jnp.float32),
        pltpu.VMEM((CHUNK, DF), jnp.float32),
        pltpu.VMEM((CHUNK,), jnp.int32),
        pltpu.VMEM((CHUNK,), jnp.int32),
        pltpu.VMEM((CHUNK,), jnp.float32),
        pltpu.SemaphoreType.DMA,
        pltpu.SemaphoreType.DMA,
    ],
)
def _spmm_pair(x0, x1, src_h, dst_h, w0_h, w1_h, o0, o1,
               acc, rows, srcb, dstb, wb, gsem, ssem):
    c = lax.axis_index("c")
    s = lax.axis_index("s")
    zero16 = jnp.zeros((16,), jnp.float32)

    def zrow(r, carry):
        for k8 in range(8):
            rows[r, pl.ds(k8 * 16, 16)] = zero16
        return carry

    lax.fori_loop(0, CHUNK, zrow, 0)

    base = s * ROWS_PT
    for i in range(5):
        pltpu.sync_copy(rows, acc.at[pl.ds(base + i * CHUNK, CHUNK)])
    plsc.subcore_barrier()

    def chunk_body(j, carry):
        row0 = s * NCHUNKS + j
        e0 = row0 * CHUNK
        pltpu.sync_copy(src_h.at[pl.ds(e0, CHUNK)], srcb)
        pltpu.sync_copy(dst_h.at[row0], dstb)

        @pl.when(c == 0)
        def _():
            pltpu.sync_copy(w0_h.at[pl.ds(e0, CHUNK)], wb)
            pltpu.async_copy(x0.at[srcb], rows, gsem).wait()

        @pl.when(c == 1)
        def _():
            pltpu.sync_copy(w1_h.at[pl.ds(e0, CHUNK)], wb)
            pltpu.async_copy(x1.at[srcb], rows, gsem).wait()

        def group(g, gcarry):
            wvec = wb[pl.ds(g * 16, 16)]
            for j in range(16):
                e = g * 16 + j
                wv = wvec[j]
                for k8 in range(8):
                    sl = pl.ds(k8 * 16, 16)
                    rows[e, sl] = rows[e, sl] * wv
            return gcarry

        lax.fori_loop(0, CHUNK // 16, group, 0)
        pltpu.async_copy(rows, acc.at[dstb], ssem, add=True).wait()
        return carry

    lax.fori_loop(0, NCHUNKS, chunk_body, 0)
    plsc.subcore_barrier()

    @pl.when(jnp.logical_and(c == 0, s < NTILES - 1))
    def _():
        pltpu.sync_copy(acc.at[pl.ds(base, ROWS_PT)], o0.at[pl.ds(base, ROWS_PT)])

    @pl.when(jnp.logical_and(c == 1, s < NTILES - 1))
    def _():
        pltpu.sync_copy(acc.at[pl.ds(base, ROWS_PT)], o1.at[pl.ds(base, ROWS_PT)])

    @pl.when(jnp.logical_and(c == 0, s == NTILES - 1))
    def _():
        pltpu.sync_copy(acc.at[pl.ds(base, LAST_ROWS)], o0.at[pl.ds(base, LAST_ROWS)])

    @pl.when(jnp.logical_and(c == 1, s == NTILES - 1))
    def _():
        pltpu.sync_copy(acc.at[pl.ds(base, LAST_ROWS)], o1.at[pl.ds(base, LAST_ROWS)])


def _soft(x, eta):
    return jax.nn.relu(x - eta) - jax.nn.relu(-x - eta)


def kernel(F, edge_index, w0_values, w1_values, d, mask):
    dst = edge_index[0]
    src = edge_index[1]
    npad = E_PAD - E
    padidx = (jnp.arange(npad, dtype=jnp.int32) * 97) % N
    src_p = jnp.concatenate([src, padidx])
    dst_p = jnp.concatenate([dst, padidx])
    zpad = jnp.zeros((npad,), jnp.float32)
    w0_p = jnp.concatenate([w0_values, zpad])
    w1_p = jnp.concatenate([w1_values, zpad])
    dst2d = dst_p.reshape(E_PAD // CHUNK, CHUNK)

    def spmm_pair(X0, X1):
        return _spmm_pair(X0, X1, src_p, dst2d, w0_p, w1_p)

    d1 = d[:, None]
    m2 = mask * mask
    c2 = 1.0 / (d1 * m2 + GAMMA)
    c1F = d1 * m2 * F

    A0, A1 = spmm_pair(F, F)
    Y1 = jnp.zeros((N, DF), jnp.float32)
    U = F
    for k in range(1, STEPS + 1):
        v1 = Y1 - _soft(A1 + Y1, d1)
        P0, P1 = spmm_pair(A0, v1)
        U = (c1F - P1 + P0) * c2
        if k < STEPS:
            B0, B1 = spmm_pair(U, U)
            Y1 = B1 + v1
            A0, A1 = B0, B1
    return U


# speedup vs baseline: 6.2243x; 2.0537x over previous
import functools

import jax
import jax.numpy as jnp
from jax import lax
from jax.experimental import pallas as pl
from jax.experimental.pallas import tpu as pltpu
from jax.experimental.pallas import tpu_sc as plsc

N = 10000
E = 320000
DF = 128
GAMMA = 1.0
STEPS = 14

NCORES = 2
NTILES = 16
CHUNK = 128
NCHUNKS = 160
SEG = 16
NBLK = NCHUNKS // SEG
EPT = NCHUNKS * CHUNK
E_PAD = EPT * NTILES
ROWS_PT = 640
ACC_N = NTILES * ROWS_PT
LAST_ROWS = N - 15 * ROWS_PT

_mesh = plsc.VectorSubcoreMesh(core_axis_name="c", subcore_axis_name="s")


@functools.partial(
    pl.kernel,
    out_type=(
        jax.ShapeDtypeStruct((N, DF), jnp.float32),
        jax.ShapeDtypeStruct((N, DF), jnp.float32),
    ),
    mesh=_mesh,
    scratch_types=[
        pltpu.VMEM_SHARED((ACC_N, DF), jnp.float32),
        pltpu.VMEM((CHUNK, DF), jnp.float32),
        pltpu.VMEM((CHUNK, DF), jnp.float32),
        pltpu.VMEM((SEG, CHUNK), jnp.int32),
        pltpu.VMEM((SEG, CHUNK), jnp.int32),
        pltpu.VMEM((SEG, CHUNK), jnp.float32),
        pltpu.SemaphoreType.DMA,
        pltpu.SemaphoreType.DMA,
        pltpu.SemaphoreType.DMA,
        pltpu.SemaphoreType.DMA,
    ],
)
def _spmm_pair(x0, x1, src_h, dst_h, w0_h, w1_h, o0, o1,
               acc, rows0, rows1, srcv, dstv, wv, gsem0, gsem1, ssem0, ssem1):
    c = lax.axis_index("c")
    s = lax.axis_index("s")
    t0 = s * NCHUNKS

    zero16 = jnp.zeros((16,), jnp.float32)

    def zrow(r, carry):
        for k8 in range(8):
            rows0[r, pl.ds(k8 * 16, 16)] = zero16
        return carry

    lax.fori_loop(0, CHUNK, zrow, 0)
    base = s * ROWS_PT
    for i in range(5):
        pltpu.sync_copy(rows0, acc.at[pl.ds(base + i * CHUNK, CHUNK)])
    plsc.subcore_barrier()

    def gather(j, buf, sem):
        @pl.when(c == 0)
        def _():
            pltpu.async_copy(x0.at[srcv.at[j]], buf, sem)

        @pl.when(c == 1)
        def _():
            pltpu.async_copy(x1.at[srcv.at[j]], buf, sem)

    def gather_wait(buf, sem):
        pltpu.make_async_copy(x0.at[srcv.at[0]], buf, sem).wait()

    def scatter(j, buf, sem):
        pltpu.async_copy(buf, acc.at[dstv.at[j]], sem, add=True)

    def scatter_wait(buf, sem):
        pltpu.make_async_copy(buf, acc.at[dstv.at[0]], sem).wait()

    def scale(j, buf):
        def group(g, gcarry):
            wvec = wv[j, pl.ds(g * 16, 16)]
            for jj in range(16):
                e = g * 16 + jj
                w = wvec[jj]
                for k8 in range(8):
                    sl = pl.ds(k8 * 16, 16)
                    buf[e, sl] = buf[e, sl] * w
            return gcarry

        lax.fori_loop(0, CHUNK // 16, group, 0)

    def block_body(b, carry):
        pltpu.sync_copy(src_h.at[pl.ds(t0 + b * SEG, SEG)], srcv)
        pltpu.sync_copy(dst_h.at[pl.ds(t0 + b * SEG, SEG)], dstv)

        @pl.when(c == 0)
        def _():
            pltpu.sync_copy(w0_h.at[pl.ds(t0 + b * SEG, SEG)], wv)

        @pl.when(c == 1)
        def _():
            pltpu.sync_copy(w1_h.at[pl.ds(t0 + b * SEG, SEG)], wv)

        gather(0, rows0, gsem0)

        def pair_body(g, pcarry):
            j0 = 2 * g
            j1 = j0 + 1

            @pl.when(g > 0)
            def _():
                scatter_wait(rows1, ssem1)

            gather(j1, rows1, gsem1)
            gather_wait(rows0, gsem0)
            scale(j0, rows0)
            scatter(j0, rows0, ssem0)
            gather_wait(rows1, gsem1)
            scale(j1, rows1)
            scatter(j1, rows1, ssem1)
            scatter_wait(rows0, ssem0)

            @pl.when(g < SEG // 2 - 1)
            def _():
                gather(j0 + 2, rows0, gsem0)

            return pcarry

        lax.fori_loop(0, SEG // 2, pair_body, 0)
        scatter_wait(rows1, ssem1)
        return carry

    lax.fori_loop(0, NBLK, block_body, 0)
    plsc.subcore_barrier()

    @pl.when(jnp.logical_and(c == 0, s < NTILES - 1))
    def _():
        pltpu.sync_copy(acc.at[pl.ds(base, ROWS_PT)], o0.at[pl.ds(base, ROWS_PT)])

    @pl.when(jnp.logical_and(c == 1, s < NTILES - 1))
    def _():
        pltpu.sync_copy(acc.at[pl.ds(base, ROWS_PT)], o1.at[pl.ds(base, ROWS_PT)])

    @pl.when(jnp.logical_and(c == 0, s == NTILES - 1))
    def _():
        pltpu.sync_copy(acc.at[pl.ds(base, LAST_ROWS)], o0.at[pl.ds(base, LAST_ROWS)])

    @pl.when(jnp.logical_and(c == 1, s == NTILES - 1))
    def _():
        pltpu.sync_copy(acc.at[pl.ds(base, LAST_ROWS)], o1.at[pl.ds(base, LAST_ROWS)])


def _soft(x, eta):
    return jax.nn.relu(x - eta) - jax.nn.relu(-x - eta)


def kernel(F, edge_index, w0_values, w1_values, d, mask):
    dst = edge_index[0]
    src = edge_index[1]
    npad = E_PAD - E
    padidx = (jnp.arange(npad, dtype=jnp.int32) * 97) % N
    src_p = jnp.concatenate([src, padidx]).reshape(E_PAD // CHUNK, CHUNK)
    dst_p = jnp.concatenate([dst, padidx]).reshape(E_PAD // CHUNK, CHUNK)
    zpad = jnp.zeros((npad,), jnp.float32)
    w0_p = jnp.concatenate([w0_values, zpad]).reshape(E_PAD // CHUNK, CHUNK)
    w1_p = jnp.concatenate([w1_values, zpad]).reshape(E_PAD // CHUNK, CHUNK)

    def spmm_pair(X0, X1):
        return _spmm_pair(X0, X1, src_p, dst_p, w0_p, w1_p)

    d1 = d[:, None]
    m2 = mask * mask
    c2 = 1.0 / (d1 * m2 + GAMMA)
    c1F = d1 * m2 * F

    A0, A1 = spmm_pair(F, F)
    Y1 = jnp.zeros((N, DF), jnp.float32)
    U = F
    for k in range(1, STEPS + 1):
        v1 = Y1 - _soft(A1 + Y1, d1)
        P0, P1 = spmm_pair(A0, v1)
        U = (c1F - P1 + P0) * c2
        if k < STEPS:
            B0, B1 = spmm_pair(U, U)
            Y1 = B1 + v1
            A0, A1 = B0, B1
    return U
